# baseline (device time: 9013 ns/iter reference)
import jax
import jax.numpy as jnp
from jax import lax
from jax.experimental import pallas as pl
from jax.experimental.pallas import tpu as pltpu

N_DEV = 4


def kernel(x):
    _, m, n = x.shape

    q = m // 4
    ORDER = (0, 2, 1, 3)

    def body(x_ref, o_ref, xv_ref, acc_ref, sum1_ref, sum2_ref,
             recv1_ref, recv2_ref, in_sem, out_sems,
             send_sems1, recv_sems1, send_sems2, recv_sems2):
        my = lax.axis_index("i")
        p1 = my ^ 1
        p2 = 3 - my
        stage1_to = {0: p1, 1: p1, 2: p2, 3: p2}
        stage2_to = {0: p2, 1: p2, 2: p1, 3: p1}

        in_copy = pltpu.make_async_copy(x_ref, xv_ref, in_sem)
        in_copy.start()

        barrier_sem = pltpu.get_barrier_semaphore()
        for p in (p1, p2):
            pl.semaphore_signal(
                barrier_sem, inc=1,
                device_id=(p,), device_id_type=pl.DeviceIdType.MESH,
            )
        in_copy.wait()
        acc_ref[...] = xv_ref[0].astype(jnp.bfloat16)
        pl.semaphore_wait(barrier_sem, 2)

        sl = {k: pl.ds(k * q, q) for k in range(4)}

        r1 = {}
        for k in ORDER:
            r1[k] = pltpu.make_async_remote_copy(
                src_ref=acc_ref.at[sl[k]],
                dst_ref=recv1_ref.at[k],
                send_sem=send_sems1.at[k],
                recv_sem=recv_sems1.at[k],
                device_id=(stage1_to[k],),
                device_id_type=pl.DeviceIdType.MESH,
            )
            r1[k].start()

        r2 = {}
        for k in ORDER:
            r1[k].wait_recv()
            sum1_ref[sl[k]] = acc_ref[sl[k]] + recv1_ref[k]
            r2[k] = pltpu.make_async_remote_copy(
                src_ref=sum1_ref.at[sl[k]],
                dst_ref=recv2_ref.at[k],
                send_sem=send_sems2.at[k],
                recv_sem=recv_sems2.at[k],
                device_id=(stage2_to[k],),
                device_id_type=pl.DeviceIdType.MESH,
            )
            r2[k].start()

        out_copies = {}
        for k in ORDER:
            r2[k].wait_recv()
            sum2_ref[sl[k]] = sum1_ref[sl[k]] + recv2_ref[k]
            out_copies[k] = pltpu.make_async_copy(
                sum2_ref.at[sl[k]], o_ref.at[sl[k]], out_sems.at[k]
            )
            out_copies[k].start()

        for k in ORDER:
            out_copies[k].wait()
            r1[k].wait_send()
            r2[k].wait_send()

    return pl.pallas_call(
        body,
        out_shape=jax.ShapeDtypeStruct((m, n), jnp.bfloat16),
        in_specs=[pl.BlockSpec(memory_space=pl.ANY)],
        out_specs=pl.BlockSpec(memory_space=pl.ANY),
        scratch_shapes=[
            pltpu.VMEM((1, m, n), jnp.float32),
            pltpu.VMEM((m, n), jnp.bfloat16),
            pltpu.VMEM((m, n), jnp.bfloat16),
            pltpu.VMEM((m, n), jnp.bfloat16),
            pltpu.VMEM((4, q, n), jnp.bfloat16),
            pltpu.VMEM((4, q, n), jnp.bfloat16),
            pltpu.SemaphoreType.DMA,
            pltpu.SemaphoreType.DMA((4,)),
            pltpu.SemaphoreType.DMA((4,)),
            pltpu.SemaphoreType.DMA((4,)),
            pltpu.SemaphoreType.DMA((4,)),
            pltpu.SemaphoreType.DMA((4,)),
        ],
        compiler_params=pltpu.CompilerParams(collective_id=0),
    )(x)


# device time: 8691 ns/iter; 1.0370x vs baseline; 1.0370x over previous
import jax
import jax.numpy as jnp
from jax import lax
from jax.experimental import pallas as pl
from jax.experimental.pallas import tpu as pltpu

N_DEV = 4


def kernel(x):
    _, m, n = x.shape

    q = m // 4
    ORDER = (0, 2, 1, 3)

    def body(x_ref, o_ref, xv_ref, acc_ref, sum1_ref, sum2_ref,
             recv1_ref, recv2_ref, in_sem, out_sems,
             send_sems1, recv_sems1, send_sems2, recv_sems2):
        my = lax.axis_index("i")
        p1 = my ^ 1
        p2 = 3 - my
        stage1_to = {0: p1, 1: p1, 2: p2, 3: p2}
        stage2_to = {0: p2, 1: p2, 2: p1, 3: p1}

        in_copy = pltpu.make_async_copy(x_ref, xv_ref, in_sem)
        in_copy.start()

        barrier_sem = pltpu.get_barrier_semaphore()
        for p in (p1, p2):
            pl.semaphore_signal(
                barrier_sem, inc=1,
                device_id=(p,), device_id_type=pl.DeviceIdType.MESH,
            )
        in_copy.wait()
        acc_ref[...] = xv_ref[0].astype(jnp.bfloat16)
        pl.semaphore_wait(barrier_sem, 2)

        sl = {k: pl.ds(k * q, q) for k in range(4)}

        r1 = {}
        for k in ORDER:
            r1[k] = pltpu.make_async_remote_copy(
                src_ref=acc_ref.at[sl[k]],
                dst_ref=recv1_ref.at[k],
                send_sem=send_sems1.at[k],
                recv_sem=recv_sems1.at[k],
                device_id=(stage1_to[k],),
                device_id_type=pl.DeviceIdType.MESH,
            )
            r1[k].start()

        r2 = {}
        for k in ORDER:
            r1[k].wait_recv()
            sum1_ref[sl[k]] = acc_ref[sl[k]] + recv1_ref[k]
            r2[k] = pltpu.make_async_remote_copy(
                src_ref=sum1_ref.at[sl[k]],
                dst_ref=recv2_ref.at[k],
                send_sem=send_sems2.at[k],
                recv_sem=recv_sems2.at[k],
                device_id=(stage2_to[k],),
                device_id_type=pl.DeviceIdType.MESH,
            )
            r2[k].start()

        out_copies = {}
        for k in ORDER:
            r2[k].wait_recv()
            sum2_ref[sl[k]] = sum1_ref[sl[k]] + recv2_ref[k]
            out_copies[k] = pltpu.make_async_copy(
                sum2_ref.at[sl[k]], o_ref.at[sl[k]], out_sems.at[k]
            )
            out_copies[k].start()

        for k in ORDER:
            out_copies[k].wait()
            r1[k].wait_send()
            r2[k].wait_send()

    x = pltpu.with_memory_space_constraint(x, pltpu.MemorySpace.HBM)
    return pl.pallas_call(
        body,
        out_shape=jax.ShapeDtypeStruct((m, n), jnp.bfloat16),
        in_specs=[pl.BlockSpec(memory_space=pl.ANY)],
        out_specs=pl.BlockSpec(memory_space=pltpu.MemorySpace.HBM),
        scratch_shapes=[
            pltpu.VMEM((1, m, n), jnp.float32),
            pltpu.VMEM((m, n), jnp.bfloat16),
            pltpu.VMEM((m, n), jnp.bfloat16),
            pltpu.VMEM((m, n), jnp.bfloat16),
            pltpu.VMEM((4, q, n), jnp.bfloat16),
            pltpu.VMEM((4, q, n), jnp.bfloat16),
            pltpu.SemaphoreType.DMA,
            pltpu.SemaphoreType.DMA((4,)),
            pltpu.SemaphoreType.DMA((4,)),
            pltpu.SemaphoreType.DMA((4,)),
            pltpu.SemaphoreType.DMA((4,)),
            pltpu.SemaphoreType.DMA((4,)),
        ],
        compiler_params=pltpu.CompilerParams(collective_id=0),
    )(x)


# device time: 8599 ns/iter; 1.0481x vs baseline; 1.0107x over previous
import jax
import jax.numpy as jnp
from jax import lax
from jax.experimental import pallas as pl
from jax.experimental.pallas import tpu as pltpu

N_DEV = 4


def kernel(x):
    _, m, n = x.shape

    q = m // 4
    ORDER = (0, 2, 1, 3)

    def body(x_ref, o_ref, xv_ref, acc_ref, sum1_ref, sum2_ref,
             recv1_ref, recv2_ref, in_sems, out_sems,
             send_sems1, recv_sems1, send_sems2, recv_sems2):
        my = lax.axis_index("i")
        p1 = my ^ 1
        p2 = 3 - my
        stage1_to = {0: p1, 1: p1, 2: p2, 3: p2}
        stage2_to = {0: p2, 1: p2, 2: p1, 3: p1}

        sl = {k: pl.ds(k * q, q) for k in range(4)}

        in_copies = {}
        for k in ORDER:
            in_copies[k] = pltpu.make_async_copy(
                x_ref.at[0, sl[k]], xv_ref.at[sl[k]], in_sems.at[k]
            )
            in_copies[k].start()

        barrier_sem = pltpu.get_barrier_semaphore()
        for p in (p1, p2):
            pl.semaphore_signal(
                barrier_sem, inc=1,
                device_id=(p,), device_id_type=pl.DeviceIdType.MESH,
            )
        pl.semaphore_wait(barrier_sem, 2)

        r1 = {}
        for k in ORDER:
            in_copies[k].wait()
            acc_ref[sl[k]] = xv_ref[sl[k]].astype(jnp.bfloat16)
            r1[k] = pltpu.make_async_remote_copy(
                src_ref=acc_ref.at[sl[k]],
                dst_ref=recv1_ref.at[k],
                send_sem=send_sems1.at[k],
                recv_sem=recv_sems1.at[k],
                device_id=(stage1_to[k],),
                device_id_type=pl.DeviceIdType.MESH,
            )
            r1[k].start()

        r2 = {}
        for k in ORDER:
            r1[k].wait_recv()
            sum1_ref[sl[k]] = acc_ref[sl[k]] + recv1_ref[k]
            r2[k] = pltpu.make_async_remote_copy(
                src_ref=sum1_ref.at[sl[k]],
                dst_ref=recv2_ref.at[k],
                send_sem=send_sems2.at[k],
                recv_sem=recv_sems2.at[k],
                device_id=(stage2_to[k],),
                device_id_type=pl.DeviceIdType.MESH,
            )
            r2[k].start()

        out_copies = {}
        for k in ORDER:
            r2[k].wait_recv()
            sum2_ref[sl[k]] = sum1_ref[sl[k]] + recv2_ref[k]
            out_copies[k] = pltpu.make_async_copy(
                sum2_ref.at[sl[k]], o_ref.at[sl[k]], out_sems.at[k]
            )
            out_copies[k].start()

        for k in ORDER:
            out_copies[k].wait()
            r1[k].wait_send()
            r2[k].wait_send()

    x = pltpu.with_memory_space_constraint(x, pltpu.MemorySpace.HBM)
    return pl.pallas_call(
        body,
        out_shape=jax.ShapeDtypeStruct((m, n), jnp.bfloat16),
        in_specs=[pl.BlockSpec(memory_space=pl.ANY)],
        out_specs=pl.BlockSpec(memory_space=pltpu.MemorySpace.HBM),
        scratch_shapes=[
            pltpu.VMEM((m, n), jnp.float32),
            pltpu.VMEM((m, n), jnp.bfloat16),
            pltpu.VMEM((m, n), jnp.bfloat16),
            pltpu.VMEM((m, n), jnp.bfloat16),
            pltpu.VMEM((4, q, n), jnp.bfloat16),
            pltpu.VMEM((4, q, n), jnp.bfloat16),
            pltpu.SemaphoreType.DMA((4,)),
            pltpu.SemaphoreType.DMA((4,)),
            pltpu.SemaphoreType.DMA((4,)),
            pltpu.SemaphoreType.DMA((4,)),
            pltpu.SemaphoreType.DMA((4,)),
            pltpu.SemaphoreType.DMA((4,)),
        ],
        compiler_params=pltpu.CompilerParams(collective_id=0),
    )(x)


# device time: 8591 ns/iter; 1.0491x vs baseline; 1.0009x over previous
import jax
import jax.numpy as jnp
from jax import lax
from jax.experimental import pallas as pl
from jax.experimental.pallas import tpu as pltpu

N_DEV = 4


def kernel(x):
    _, m, n = x.shape

    NC = 8
    q = m // NC
    ORDER = tuple(
        k for pair in zip(range(NC // 2), range(NC // 2, NC)) for k in pair
    )

    def body(x_ref, o_ref, xv_ref, acc_ref, sum1_ref, sum2_ref,
             recv1_ref, recv2_ref, in_sems, out_sems,
             send_sems1, recv_sems1, send_sems2, recv_sems2):
        my = lax.axis_index("i")
        p1 = my ^ 1
        p2 = 3 - my
        stage1_to = {k: (p1 if k < NC // 2 else p2) for k in range(NC)}
        stage2_to = {k: (p2 if k < NC // 2 else p1) for k in range(NC)}

        sl = {k: pl.ds(k * q, q) for k in range(NC)}

        in_copies = {}
        for k in ORDER:
            in_copies[k] = pltpu.make_async_copy(
                x_ref.at[0, sl[k]], xv_ref.at[sl[k]], in_sems.at[k]
            )
            in_copies[k].start()

        barrier_sem = pltpu.get_barrier_semaphore()
        for p in (p1, p2):
            pl.semaphore_signal(
                barrier_sem, inc=1,
                device_id=(p,), device_id_type=pl.DeviceIdType.MESH,
            )
        pl.semaphore_wait(barrier_sem, 2)

        r1 = {}
        for k in ORDER:
            in_copies[k].wait()
            acc_ref[sl[k]] = xv_ref[sl[k]].astype(jnp.bfloat16)
            r1[k] = pltpu.make_async_remote_copy(
                src_ref=acc_ref.at[sl[k]],
                dst_ref=recv1_ref.at[k],
                send_sem=send_sems1.at[k],
                recv_sem=recv_sems1.at[k],
                device_id=(stage1_to[k],),
                device_id_type=pl.DeviceIdType.MESH,
            )
            r1[k].start()

        r2 = {}
        for k in ORDER:
            r1[k].wait_recv()
            sum1_ref[sl[k]] = acc_ref[sl[k]] + recv1_ref[k]
            r2[k] = pltpu.make_async_remote_copy(
                src_ref=sum1_ref.at[sl[k]],
                dst_ref=recv2_ref.at[k],
                send_sem=send_sems2.at[k],
                recv_sem=recv_sems2.at[k],
                device_id=(stage2_to[k],),
                device_id_type=pl.DeviceIdType.MESH,
            )
            r2[k].start()

        out_copies = {}
        for k in ORDER:
            r2[k].wait_recv()
            sum2_ref[sl[k]] = sum1_ref[sl[k]] + recv2_ref[k]
            out_copies[k] = pltpu.make_async_copy(
                sum2_ref.at[sl[k]], o_ref.at[sl[k]], out_sems.at[k]
            )
            out_copies[k].start()

        for k in ORDER:
            out_copies[k].wait()
            r1[k].wait_send()
            r2[k].wait_send()

    x = pltpu.with_memory_space_constraint(x, pltpu.MemorySpace.HBM)
    return pl.pallas_call(
        body,
        out_shape=jax.ShapeDtypeStruct((m, n), jnp.bfloat16),
        in_specs=[pl.BlockSpec(memory_space=pl.ANY)],
        out_specs=pl.BlockSpec(memory_space=pltpu.MemorySpace.HBM),
        scratch_shapes=[
            pltpu.VMEM((m, n), jnp.float32),
            pltpu.VMEM((m, n), jnp.bfloat16),
            pltpu.VMEM((m, n), jnp.bfloat16),
            pltpu.VMEM((m, n), jnp.bfloat16),
            pltpu.VMEM((NC, q, n), jnp.bfloat16),
            pltpu.VMEM((NC, q, n), jnp.bfloat16),
            pltpu.SemaphoreType.DMA((NC,)),
            pltpu.SemaphoreType.DMA((NC,)),
            pltpu.SemaphoreType.DMA((NC,)),
            pltpu.SemaphoreType.DMA((NC,)),
            pltpu.SemaphoreType.DMA((NC,)),
            pltpu.SemaphoreType.DMA((NC,)),
        ],
        compiler_params=pltpu.CompilerParams(collective_id=0),
    )(x)


# device time: 8525 ns/iter; 1.0572x vs baseline; 1.0077x over previous
import jax
import jax.numpy as jnp
from jax import lax
from jax.experimental import pallas as pl
from jax.experimental.pallas import tpu as pltpu

N_DEV = 4


def kernel(x):
    _, m, n = x.shape

    NC = 4
    q = m // NC
    ORDER = tuple(
        k for pair in zip(range(NC // 2), range(NC // 2, NC)) for k in pair
    )

    def body(x_ref, o_ref, xv_ref, acc_ref, sum1_ref,
             recv1_ref, recv2_ref, in_sems,
             send_sems1, recv_sems1, send_sems2, recv_sems2):
        my = lax.axis_index("i")
        p1 = my ^ 1
        p2 = 3 - my
        stage1_to = {k: (p1 if k < NC // 2 else p2) for k in range(NC)}
        stage2_to = {k: (p2 if k < NC // 2 else p1) for k in range(NC)}

        sl = {k: pl.ds(k * q, q) for k in range(NC)}

        in_copies = {}
        for k in ORDER:
            in_copies[k] = pltpu.make_async_copy(
                x_ref.at[0, sl[k]], xv_ref.at[sl[k]], in_sems.at[k]
            )
            in_copies[k].start()

        barrier_sem = pltpu.get_barrier_semaphore()
        for p in (p1, p2):
            pl.semaphore_signal(
                barrier_sem, inc=1,
                device_id=(p,), device_id_type=pl.DeviceIdType.MESH,
            )
        pl.semaphore_wait(barrier_sem, 2)

        r1 = {}
        for k in ORDER:
            in_copies[k].wait()
            acc_ref[sl[k]] = xv_ref[sl[k]].astype(jnp.bfloat16)
            r1[k] = pltpu.make_async_remote_copy(
                src_ref=acc_ref.at[sl[k]],
                dst_ref=recv1_ref.at[k],
                send_sem=send_sems1.at[k],
                recv_sem=recv_sems1.at[k],
                device_id=(stage1_to[k],),
                device_id_type=pl.DeviceIdType.MESH,
            )
            r1[k].start()

        r2 = {}
        for k in ORDER:
            r1[k].wait_recv()
            sum1_ref[sl[k]] = acc_ref[sl[k]] + recv1_ref[k]
            r2[k] = pltpu.make_async_remote_copy(
                src_ref=sum1_ref.at[sl[k]],
                dst_ref=recv2_ref.at[k],
                send_sem=send_sems2.at[k],
                recv_sem=recv_sems2.at[k],
                device_id=(stage2_to[k],),
                device_id_type=pl.DeviceIdType.MESH,
            )
            r2[k].start()

        for k in ORDER:
            r2[k].wait_recv()
            o_ref[sl[k]] = sum1_ref[sl[k]] + recv2_ref[k]

        for k in ORDER:
            r1[k].wait_send()
            r2[k].wait_send()

    x = pltpu.with_memory_space_constraint(x, pltpu.MemorySpace.HBM)
    return pl.pallas_call(
        body,
        out_shape=jax.ShapeDtypeStruct((m, n), jnp.bfloat16),
        in_specs=[pl.BlockSpec(memory_space=pl.ANY)],
        out_specs=pl.BlockSpec(memory_space=pltpu.MemorySpace.VMEM),
        scratch_shapes=[
            pltpu.VMEM((m, n), jnp.float32),
            pltpu.VMEM((m, n), jnp.bfloat16),
            pltpu.VMEM((m, n), jnp.bfloat16),
            pltpu.VMEM((NC, q, n), jnp.bfloat16),
            pltpu.VMEM((NC, q, n), jnp.bfloat16),
            pltpu.SemaphoreType.DMA((NC,)),
            pltpu.SemaphoreType.DMA((NC,)),
            pltpu.SemaphoreType.DMA((NC,)),
            pltpu.SemaphoreType.DMA((NC,)),
            pltpu.SemaphoreType.DMA((NC,)),
        ],
        compiler_params=pltpu.CompilerParams(collective_id=0),
    )(x)
